# baseline (speedup 1.0000x reference)
import functools

import jax
import jax.numpy as jnp
from jax import lax
from jax.experimental import pallas as pl
from jax.experimental.pallas import tpu as pltpu
from jax.experimental.pallas import tpu_sc as plsc

_EPS = 1e-07
_N = 1_000_000
_C = 100
_D = 64
_B = 16384
_NP = _N // 2
_EP = _B // 2

_NW = 32
_W_OWN = _N // _NW
_W_PAD = 31744
_SEG = 3968
_NSEG = _W_PAD // _SEG
_BPW = _B // _NW


def _bcast16(x):
  lane = lax.iota(jnp.int32, 16)
  return jnp.sum(jnp.where(lane == 0, x, 0)) + jnp.zeros((16,), x.dtype)


def _g16(x, ids):
  dnums = lax.GatherDimensionNumbers(
      offset_dims=(), collapsed_slice_dims=(0,), start_index_map=(0,))
  return lax.gather(x, ids[:, None], dnums, (1,),
                    mode=lax.GatherScatterMode.PROMISE_IN_BOUNDS)


def _decode(tg):
  bits = jnp.bitwise_and(tg, 65535)
  fb = bits.astype(jnp.float32)
  hb = lax.shift_right_arithmetic(plsc.bitcast(fb, jnp.int32), 23) - 127
  j = (lax.shift_right_arithmetic(tg, 16) - 1) * 16 + hb
  return jnp.clip(j, 0, _B - 1)


def _sc_body(idx_hbm, emb_hbm,
             buf_hbm,
             idx_v, tag_v, p2d, s2d, js2d, jo2d,
             chp, chs, cho, chbuf, ebufs, ebufo,
             sem_g):
  wid = lax.axis_index("s") * 2 + lax.axis_index("c")
  lo = wid * _W_OWN
  lane = lax.iota(jnp.int32, 16)

  pltpu.sync_copy(idx_hbm, idx_v)

  def _zero(i, _):
    tag_v[pl.ds(i * 16, 16)] = jnp.zeros((16,), jnp.int32)
    return 0
  lax.fori_loop(0, _W_PAD // 16, _zero, 0)

  lane_bit = lax.shift_left(jnp.ones((16,), jnp.int32), lane)
  def _scan(i, _):
    v = idx_v[pl.ds(i * 16, 16)]
    m = jnp.logical_and(v >= lo, v < lo + _W_OWN)
    off = jnp.where(m, v - lo, 0)
    claim = (i + 1) * 65536 + jnp.zeros((16,), jnp.int32)
    plsc.store_scatter(tag_v, [off], claim, mask=m)
    plsc.addupdate_scatter(tag_v, [off], lane_bit, mask=m)
    return 0
  lax.fori_loop(0, _B // 16, _scan, 0)

  def _seg(s, _):
    seg_base = s * _SEG

    def _compact(t, c):
      off = seg_base + t * 16
      tg = tag_v[pl.ds(off, 16)]
      ntg = _g16(tg, jnp.bitwise_xor(lane, 1))
      hm = tg > 0
      hmi = hm.astype(jnp.int32)
      pos = c + plsc.cumsum(hmi) - 1
      posm = jnp.where(hm, pos, 0)
      pr = lax.shift_right_arithmetic(posm, 7)
      pc = jnp.bitwise_and(posm, 127)
      rowv = lo + off + lane
      pv = lax.shift_right_arithmetic(rowv, 1)
      sv = jnp.bitwise_and(rowv, 1)
      js = _decode(tg)
      jo = jnp.where(ntg > 0, _decode(ntg), -1)
      plsc.store_scatter(p2d, [pr, pc], pv, mask=hm)
      plsc.store_scatter(s2d, [pr, pc], sv, mask=hm)
      plsc.store_scatter(js2d, [pr, pc], js, mask=hm)
      plsc.store_scatter(jo2d, [pr, pc], jo, mask=hm)
      return c + jnp.sum(hmi)
    c = lax.fori_loop(0, _SEG // 16, _compact, 0)

    nch = lax.shift_right_arithmetic(c + 127, 7)

    def _chunk_body(k):
      rem = jnp.minimum(c - k * 128, 128)
      e0p = _bcast16(p2d[k, pl.ds(0, 16)])
      e0s = _bcast16(s2d[k, pl.ds(0, 16)])
      e0js = _bcast16(js2d[k, pl.ds(0, 16)])
      e0jo = _bcast16(jo2d[k, pl.ds(0, 16)])
      for sub in range(8):
        p16 = lane + sub * 16
        valid = p16 < rem
        cp = jnp.where(valid, p2d[k, pl.ds(sub * 16, 16)], e0p)
        cs = jnp.where(valid, s2d[k, pl.ds(sub * 16, 16)], e0s)
        cjs = jnp.where(valid, js2d[k, pl.ds(sub * 16, 16)], e0js)
        cjo = jnp.where(valid, jo2d[k, pl.ds(sub * 16, 16)], e0jo)
        p2d[k, pl.ds(sub * 16, 16)] = cp
        s2d[k, pl.ds(sub * 16, 16)] = cs
        js2d[k, pl.ds(sub * 16, 16)] = cjs
        jo2d[k, pl.ds(sub * 16, 16)] = cjo
        chp[pl.ds(sub * 16, 16)] = cp
        chs[pl.ds(sub * 16, 16)] = lax.shift_right_arithmetic(cjs, 1)
        cho[pl.ds(sub * 16, 16)] = lax.shift_right_arithmetic(
            jnp.where(cjo >= 0, cjo, cjs), 1)
      pltpu.async_copy(buf_hbm.at[chp], chbuf, sem_g).wait()
      pltpu.async_copy(emb_hbm.at[chs], ebufs, sem_g).wait()
      pltpu.async_copy(emb_hbm.at[cho], ebufo, sem_g).wait()

      def _og(g, _):
        rows16 = lane + g * 16
        s16 = s2d[k, pl.ds(g * 16, 16)]
        js16 = js2d[k, pl.ds(g * 16, 16)]
        jo16 = jo2d[k, pl.ds(g * 16, 16)]
        m_o = jo16 >= 0
        col_s = jnp.bitwise_and(js16, 1) * 64
        col_o = jnp.bitwise_and(jnp.where(m_o, jo16, 0), 1) * 64
        side_s = s16 * 64
        side_o = (1 - s16) * 64

        def _oe(e, _2):
          ev = e + jnp.zeros((16,), jnp.int32)
          m_self = jnp.logical_and(ev >= side_s, ev < side_s + 64)
          o_self = jnp.where(m_self, ev - side_s, 0)
          vs = plsc.load_gather(ebufs, [rows16, col_s + o_self])
          m_oth = jnp.logical_and(jnp.logical_not(m_self), m_o)
          o_oth = jnp.where(m_oth, ev - side_o, 0)
          vo = plsc.load_gather(ebufo, [rows16, col_o + o_oth])
          cur = plsc.load_gather(chbuf, [rows16, ev])
          nv = jnp.where(m_self, vs, jnp.where(m_oth, vo, cur))
          plsc.store_scatter(chbuf, [rows16, ev], nv)
          return 0
        lax.fori_loop(0, 128, _oe, 0)
        return 0
      lax.fori_loop(0, 8, _og, 0)

      pltpu.async_copy(chbuf, buf_hbm.at[chp], sem_g).wait()

    def _chunk(k, _):
      @pl.when(k < nch)
      def _():
        _chunk_body(k)
      return 0
    lax.fori_loop(0, _SEG // 128, _chunk, 0)
    return 0
  lax.fori_loop(0, _NSEG, _seg, 0)


def _sc_u_body(idx_hbm, u_hbm,
               ubatch_hbm,
               idx_v, ubuf, chsrc, ubuf16, sem_g):
  wid = lax.axis_index("s") * 2 + lax.axis_index("c")
  lane = lax.iota(jnp.int32, 16)
  base_b = wid * _BPW
  pltpu.sync_copy(idx_hbm.at[pl.ds(base_b, _BPW)], idx_v)
  for k in range(_BPW // 128):
    for sub in range(8):
      v = idx_v[pl.ds(k * 128 + sub * 16, 16)]
      chsrc[pl.ds(sub * 16, 16)] = lax.shift_right_arithmetic(v, 4)
    pltpu.async_copy(u_hbm.at[chsrc], ubuf16, sem_g).wait()
    for sub in range(8):
      v = idx_v[pl.ds(k * 128 + sub * 16, 16)]
      col = jnp.bitwise_and(v, 15)
      row_local = lane + sub * 16
      val = plsc.load_gather(ubuf16, [row_local, col])
      plsc.store_scatter(ubuf, [k * 128 + row_local, col * 0], val)
  pltpu.sync_copy(ubuf, ubatch_hbm.at[pl.ds(base_b, _BPW)])


def _sc_scatter(idx, emb128, buf_ref):
  mesh = plsc.VectorSubcoreMesh(core_axis_name="c", subcore_axis_name="s")
  f = functools.partial(
      pl.kernel,
      out_type=(),
      mesh=mesh,
      compiler_params=pltpu.CompilerParams(needs_layout_passes=False),
      scratch_types=[
          pltpu.VMEM((_B,), jnp.int32),
          pltpu.VMEM((_W_PAD,), jnp.int32),
          pltpu.VMEM((_SEG // 128, 128), jnp.int32),
          pltpu.VMEM((_SEG // 128, 128), jnp.int32),
          pltpu.VMEM((_SEG // 128, 128), jnp.int32),
          pltpu.VMEM((_SEG // 128, 128), jnp.int32),
          pltpu.VMEM((128,), jnp.int32),
          pltpu.VMEM((128,), jnp.int32),
          pltpu.VMEM((128,), jnp.int32),
          pltpu.VMEM((128, 128), jnp.float32),
          pltpu.VMEM((128, 128), jnp.float32),
          pltpu.VMEM((128, 128), jnp.float32),
          pltpu.SemaphoreType.DMA,
      ],
  )(_sc_body)
  return f(idx, emb128, buf_ref)


def _sc_u_gather(idx, u16):
  mesh = plsc.VectorSubcoreMesh(core_axis_name="c", subcore_axis_name="s")
  f = functools.partial(
      pl.kernel,
      out_type=[jax.ShapeDtypeStruct((_B, 1), jnp.float32)],
      mesh=mesh,
      compiler_params=pltpu.CompilerParams(
          needs_layout_passes=False, use_tc_tiling_on_sc=False),
      scratch_types=[
          pltpu.VMEM((_BPW,), jnp.int32),
          pltpu.VMEM((_BPW, 1), jnp.float32),
          pltpu.VMEM((128,), jnp.int32),
          pltpu.VMEM((128, 16), jnp.float32),
          pltpu.SemaphoreType.DMA,
      ],
  )(_sc_u_body)
  (ubatch,) = f(idx, u16)
  return ubatch


def _loss_body(acc_ref, logits_ref, labels_ref, emb_ref, u_ref,
               cent_ref, o1_ref, o2_ref, o3_ref):
  pid = pl.program_id(0)
  ac = acc_ref[0, 0]

  lg = logits_ref[...]
  lb = labels_ref[...]
  em = emb_ref[...]
  uu = u_ref[...]
  cen = cent_ref[...]

  cn = cen / (jnp.sqrt(jnp.sum(cen * cen, axis=1, keepdims=True)) + _EPS)
  en = em / (jnp.sqrt(jnp.sum(em * em, axis=1, keepdims=True)) + _EPS)
  s = lax.dot_general(en, cn, (((1,), (1,)), ((), ())),
                      preferred_element_type=jnp.float32)
  sm = jnp.max(s, axis=1, keepdims=True)
  se = jnp.exp(s - sm)
  soft = se / jnp.sum(se, axis=1, keepdims=True)

  ml = lg + ac * uu * lb
  mm = jnp.max(ml, axis=1, keepdims=True)
  lse = jnp.log(jnp.sum(jnp.exp(ml - mm), axis=1, keepdims=True))
  log_probs = ml - mm - lse
  l1p = -jnp.sum(soft * log_probs)

  mx = jnp.max(lg, axis=1, keepdims=True)
  pred = (lg == mx).astype(jnp.float32)
  term = pred + uu * lb - lb
  l2p = jnp.sum(term * term)

  m2 = jnp.max(lg, axis=1, keepdims=True)
  e2 = jnp.exp(lg - m2)
  p_true = jnp.sum(e2 * lb, axis=1, keepdims=True) / jnp.sum(
      e2, axis=1, keepdims=True)
  p_true = jnp.clip(p_true, _EPS, 1.0 - _EPS)
  u_c = jnp.clip(uu, _EPS, 1.0 - _EPS)
  u_t = jnp.clip(1.0 / (1.0 + u_c), _EPS, 1.0 - _EPS)
  t1 = p_true * (jnp.log(p_true) - jnp.log(u_t))
  t2 = (1.0 - p_true) * (jnp.log(1.0 - p_true) - jnp.log(1.0 - u_t))
  l3p = jnp.sum(t1 + t2) * (1.0 - ac)

  @pl.when(pid == 0)
  def _():
    o1_ref[0, 0] = 0.0
    o2_ref[0, 0] = 0.0
    o3_ref[0, 0] = 0.0
  o1_ref[0, 0] += l1p
  o2_ref[0, 0] += l2p
  o3_ref[0, 0] += l3p


def _loss_pallas(acc11, logits, labels, emb, u_batch, cents):
  bs = 2048
  grid = (_B // bs,)
  return pl.pallas_call(
      _loss_body,
      grid=grid,
      in_specs=[
          pl.BlockSpec(memory_space=pltpu.SMEM),
          pl.BlockSpec((bs, _C), lambda i: (i, 0)),
          pl.BlockSpec((bs, _C), lambda i: (i, 0)),
          pl.BlockSpec((bs, _D), lambda i: (i, 0)),
          pl.BlockSpec((bs, 1), lambda i: (i, 0)),
          pl.BlockSpec((_C, _D), lambda i: (0, 0)),
      ],
      out_specs=[
          pl.BlockSpec(memory_space=pltpu.SMEM),
          pl.BlockSpec(memory_space=pltpu.SMEM),
          pl.BlockSpec(memory_space=pltpu.SMEM),
      ],
      out_shape=[
          jax.ShapeDtypeStruct((1, 1), jnp.float32),
          jax.ShapeDtypeStruct((1, 1), jnp.float32),
          jax.ShapeDtypeStruct((1, 1), jnp.float32),
      ],
  )(acc11, logits, labels, emb, u_batch, cents)


def kernel(batch_original_indices, gnn_logits_batch, true_labels_batch_one_hot,
           gnn_embeddings_batch, atrain_overall_accuracy, u,
           prev_gnn_embeddings, class_centroids):
  idx = batch_original_indices.astype(jnp.int32)
  buf_ref = jax.new_ref(prev_gnn_embeddings.reshape(_NP, 128))
  _sc_scatter(idx, gnn_embeddings_batch.reshape(_EP, 128), buf_ref)
  u_batch = _sc_u_gather(idx, u.reshape(_N // 16, 16))
  new_prev = buf_ref[...].reshape(_N, _D)
  acc11 = atrain_overall_accuracy.reshape(1, 1)
  l1s, l2s, l3s = _loss_pallas(
      acc11, gnn_logits_batch, true_labels_batch_one_hot,
      gnn_embeddings_batch, u_batch, class_centroids)
  total = (l1s[0, 0] / _B + l2s[0, 0] / (_B * _C)
           + 0.5 * l3s[0, 0] / _B).reshape(1)
  return total, new_prev


# speedup vs baseline: 1.2789x; 1.2789x over previous
import functools

import jax
import jax.numpy as jnp
from jax import lax
from jax.experimental import pallas as pl
from jax.experimental.pallas import tpu as pltpu
from jax.experimental.pallas import tpu_sc as plsc

_EPS = 1e-07
_N = 1_000_000
_C = 100
_D = 64
_B = 16384

_NW = 32
_W_OWN = _N // _NW
_W_PAD = 31744
_SEG = 7936
_NSEG = _W_PAD // _SEG
_BPW = _B // _NW
_W_CP = 31248
_CSEG = 256
_NCSEG = _W_CP // _CSEG
_CTAIL = _W_CP - _NCSEG * _CSEG
_CREM = _N - _NW * _W_CP


def _bcast16(x):
  lane = lax.iota(jnp.int32, 16)
  return jnp.sum(jnp.where(lane == 0, x, 0)) + jnp.zeros((16,), x.dtype)


def _sc_body(idx_hbm, emb_hbm, u_hbm,
             out_hbm,
             ubatch_hbm,
             idx_v, tag_v, src2d, dst2d, ubuf,
             chsrc, chdst, chbuf, ubuf16,
             sem_g):
  wid = lax.axis_index("s") * 2 + lax.axis_index("c")
  lo = wid * _W_OWN
  lane = lax.iota(jnp.int32, 16)

  pltpu.sync_copy(idx_hbm, idx_v)

  base_b = wid * _BPW
  for k in range(_BPW // 128):
    for sub in range(8):
      v = idx_v[pl.ds(base_b + k * 128 + sub * 16, 16)]
      chsrc[pl.ds(sub * 16, 16)] = lax.shift_right_arithmetic(v, 4)
    pltpu.async_copy(u_hbm.at[chsrc], ubuf16, sem_g).wait()
    for sub in range(8):
      v = idx_v[pl.ds(base_b + k * 128 + sub * 16, 16)]
      col = jnp.bitwise_and(v, 15)
      row_local = lane + sub * 16
      val = plsc.load_gather(ubuf16, [row_local, col])
      plsc.store_scatter(ubuf, [k * 128 + row_local, col * 0], val)
  pltpu.sync_copy(ubuf, ubatch_hbm.at[pl.ds(base_b, _BPW)])

  def _zero(i, _):
    tag_v[pl.ds(i * 16, 16)] = jnp.zeros((16,), jnp.int32)
    return 0
  lax.fori_loop(0, _W_PAD // 16, _zero, 0)

  lane_bit = lax.shift_left(jnp.ones((16,), jnp.int32), lane)
  def _scan(i, _):
    v = idx_v[pl.ds(i * 16, 16)]
    m = jnp.logical_and(v >= lo, v < lo + _W_OWN)
    off = jnp.where(m, v - lo, 0)
    claim = (i + 1) * 65536 + jnp.zeros((16,), jnp.int32)
    plsc.store_scatter(tag_v, [off], claim, mask=m)
    plsc.addupdate_scatter(tag_v, [off], lane_bit, mask=m)
    return 0
  lax.fori_loop(0, _B // 16, _scan, 0)

  def _seg(s, _):
    seg_base = s * _SEG

    def _compact(t, c):
      off = seg_base + t * 16
      tg = tag_v[pl.ds(off, 16)]
      hm = tg > 0
      hmi = hm.astype(jnp.int32)
      pos = c + plsc.cumsum(hmi) - 1
      posm = jnp.where(hm, pos, 0)
      pr = lax.shift_right_arithmetic(posm, 7)
      pc = jnp.bitwise_and(posm, 127)
      rowv = lo + off + lane
      bits = jnp.bitwise_and(tg, 65535)
      fb = bits.astype(jnp.float32)
      hb = lax.shift_right_arithmetic(
          plsc.bitcast(fb, jnp.int32), 23) - 127
      srcj = (lax.shift_right_arithmetic(tg, 16) - 1) * 16 + hb
      srcj = jnp.clip(srcj, 0, _B - 1)
      plsc.store_scatter(dst2d, [pr, pc], rowv, mask=hm)
      plsc.store_scatter(src2d, [pr, pc], srcj, mask=hm)
      return c + jnp.sum(hmi)
    c = lax.fori_loop(0, _SEG // 16, _compact, 0)

    nch = lax.shift_right_arithmetic(c + 127, 7)

    def _chunk_body(k):
      rem = jnp.minimum(c - k * 128, 128)
      e0s = _bcast16(src2d[k, pl.ds(0, 16)])
      e0d = _bcast16(dst2d[k, pl.ds(0, 16)])
      for sub in range(8):
        p16 = lane + sub * 16
        valid = p16 < rem
        cs = src2d[k, pl.ds(sub * 16, 16)]
        cd = dst2d[k, pl.ds(sub * 16, 16)]
        chsrc[pl.ds(sub * 16, 16)] = jnp.where(valid, cs, e0s)
        chdst[pl.ds(sub * 16, 16)] = jnp.where(valid, cd, e0d)
      pltpu.async_copy(emb_hbm.at[chsrc], chbuf, sem_g).wait()
      pltpu.async_copy(chbuf, out_hbm.at[chdst], sem_g).wait()

    def _chunk(k, _):
      @pl.when(k < nch)
      def _():
        _chunk_body(k)
      return 0
    lax.fori_loop(0, _SEG // 128, _chunk, 0)
    return 0
  lax.fori_loop(0, _NSEG, _seg, 0)


def _sc_scatter_gather(idx, emb, u, prev):
  mesh = plsc.VectorSubcoreMesh(core_axis_name="c", subcore_axis_name="s")
  f = functools.partial(
      pl.kernel,
      out_type=[
          jax.ShapeDtypeStruct((_B, 1), jnp.float32),
      ],
      mesh=mesh,
      compiler_params=pltpu.CompilerParams(
          needs_layout_passes=False, use_tc_tiling_on_sc=False),
      scratch_types=[
          pltpu.VMEM((_B,), jnp.int32),
          pltpu.VMEM((_W_PAD,), jnp.int32),
          pltpu.VMEM((_SEG // 128, 128), jnp.int32),
          pltpu.VMEM((_SEG // 128, 128), jnp.int32),
          pltpu.VMEM((_BPW, 1), jnp.float32),
          pltpu.VMEM((128,), jnp.int32),
          pltpu.VMEM((128,), jnp.int32),
          pltpu.VMEM((128, _D), jnp.float32),
          pltpu.VMEM((128, 16), jnp.float32),
          pltpu.SemaphoreType.DMA,
      ],
  )(_sc_body)
  buf_ref = jax.new_ref(prev)
  (ubatch,) = f(idx, emb, u, buf_ref)
  return buf_ref[...], ubatch


def _loss_body(acc_ref, logits_ref, labels_ref, emb_ref, u_ref,
               cent_ref, o1_ref, o2_ref, o3_ref):
  pid = pl.program_id(0)
  ac = acc_ref[0, 0]

  lg = logits_ref[...]
  lb = labels_ref[...]
  em = emb_ref[...]
  uu = u_ref[...]
  cen = cent_ref[...]

  cn = cen / (jnp.sqrt(jnp.sum(cen * cen, axis=1, keepdims=True)) + _EPS)
  en = em / (jnp.sqrt(jnp.sum(em * em, axis=1, keepdims=True)) + _EPS)
  s = lax.dot_general(en, cn, (((1,), (1,)), ((), ())),
                      preferred_element_type=jnp.float32)
  sm = jnp.max(s, axis=1, keepdims=True)
  se = jnp.exp(s - sm)
  soft = se / jnp.sum(se, axis=1, keepdims=True)

  ml = lg + ac * uu * lb
  mm = jnp.max(ml, axis=1, keepdims=True)
  lse = jnp.log(jnp.sum(jnp.exp(ml - mm), axis=1, keepdims=True))
  log_probs = ml - mm - lse
  l1p = -jnp.sum(soft * log_probs)

  mx = jnp.max(lg, axis=1, keepdims=True)
  pred = (lg == mx).astype(jnp.float32)
  term = pred + uu * lb - lb
  l2p = jnp.sum(term * term)

  m2 = jnp.max(lg, axis=1, keepdims=True)
  e2 = jnp.exp(lg - m2)
  p_true = jnp.sum(e2 * lb, axis=1, keepdims=True) / jnp.sum(
      e2, axis=1, keepdims=True)
  p_true = jnp.clip(p_true, _EPS, 1.0 - _EPS)
  u_c = jnp.clip(uu, _EPS, 1.0 - _EPS)
  u_t = jnp.clip(1.0 / (1.0 + u_c), _EPS, 1.0 - _EPS)
  t1 = p_true * (jnp.log(p_true) - jnp.log(u_t))
  t2 = (1.0 - p_true) * (jnp.log(1.0 - p_true) - jnp.log(1.0 - u_t))
  l3p = jnp.sum(t1 + t2) * (1.0 - ac)

  @pl.when(pid == 0)
  def _():
    o1_ref[0, 0] = 0.0
    o2_ref[0, 0] = 0.0
    o3_ref[0, 0] = 0.0
  o1_ref[0, 0] += l1p
  o2_ref[0, 0] += l2p
  o3_ref[0, 0] += l3p


def _loss_pallas(acc11, logits, labels, emb, u_batch, cents):
  bs = 2048
  grid = (_B // bs,)
  return pl.pallas_call(
      _loss_body,
      grid=grid,
      in_specs=[
          pl.BlockSpec(memory_space=pltpu.SMEM),
          pl.BlockSpec((bs, _C), lambda i: (i, 0)),
          pl.BlockSpec((bs, _C), lambda i: (i, 0)),
          pl.BlockSpec((bs, _D), lambda i: (i, 0)),
          pl.BlockSpec((bs, 1), lambda i: (i, 0)),
          pl.BlockSpec((_C, _D), lambda i: (0, 0)),
      ],
      out_specs=[
          pl.BlockSpec(memory_space=pltpu.SMEM),
          pl.BlockSpec(memory_space=pltpu.SMEM),
          pl.BlockSpec(memory_space=pltpu.SMEM),
      ],
      out_shape=[
          jax.ShapeDtypeStruct((1, 1), jnp.float32),
          jax.ShapeDtypeStruct((1, 1), jnp.float32),
          jax.ShapeDtypeStruct((1, 1), jnp.float32),
      ],
  )(acc11, logits, labels, emb, u_batch, cents)


def kernel(batch_original_indices, gnn_logits_batch, true_labels_batch_one_hot,
           gnn_embeddings_batch, atrain_overall_accuracy, u,
           prev_gnn_embeddings, class_centroids):
  idx = batch_original_indices.astype(jnp.int32)
  new_prev, u_batch = _sc_scatter_gather(
      idx, gnn_embeddings_batch, u.reshape(_N // 16, 16), prev_gnn_embeddings)
  acc11 = atrain_overall_accuracy.reshape(1, 1)
  l1s, l2s, l3s = _loss_pallas(
      acc11, gnn_logits_batch, true_labels_batch_one_hot,
      gnn_embeddings_batch, u_batch, class_centroids)
  total = (l1s[0, 0] / _B + l2s[0, 0] / (_B * _C)
           + 0.5 * l3s[0, 0] / _B).reshape(1)
  return total, new_prev
